# baseline (speedup 1.0000x reference)
import functools

import jax
import jax.numpy as jnp
from jax import lax
from jax.experimental import pallas as pl
from jax.experimental.pallas import tpu as pltpu
from jax.experimental.pallas import tpu_sc as plsc

NUM_WORKERS = 32
LANES_PER_W = 512
SGROUP = 8
NQ = LANES_PER_W // 128


def _emb_lookup_t(loc_t, tab_flat, s_dim, c_dim, b_dim):
    ct, ci = c_dim // 8, 8
    bt, bi = b_dim // 128, 128
    n_groups = s_dim // SGROUP
    mesh = plsc.VectorSubcoreMesh(core_axis_name="c", subcore_axis_name="s")

    @functools.partial(
        pl.kernel,
        mesh=mesh,
        out_type=jax.ShapeDtypeStruct((s_dim, ct, bt, ci, bi), jnp.float32),
        scratch_types=[
            pltpu.VMEM(tab_flat.shape, jnp.float32),
            [pltpu.VMEM((SGROUP, LANES_PER_W), jnp.int32) for _ in range(2)],
            [pltpu.VMEM((ct, 1, ci, bi), jnp.float32) for _ in range(2)],
            pltpu.SemaphoreType.DMA,
            [pltpu.SemaphoreType.DMA for _ in range(2)],
            [pltpu.SemaphoreType.DMA for _ in range(2)],
        ],
        compiler_params=pltpu.CompilerParams(needs_layout_passes=False),
    )
    def k(tab_hbm, loc_hbm, out_hbm, tab_v, idx_v, buf, sem_t, sem_i, sem_s):
        w = lax.axis_index("s") * 2 + lax.axis_index("c")
        b0 = w * LANES_PER_W
        pltpu.async_copy(tab_hbm, tab_v, sem_t)
        pltpu.async_copy(loc_hbm.at[pl.ds(0, SGROUP), pl.ds(b0, LANES_PER_W)],
                         idx_v[0], sem_i[0])
        pltpu.make_async_copy(tab_hbm, tab_v, sem_t).wait()

        def do_group(g, p):
            pltpu.make_async_copy(
                loc_hbm.at[pl.ds(0, SGROUP), pl.ds(b0, LANES_PER_W)],
                idx_v[p], sem_i[p]).wait()

            @pl.when(g < n_groups - 1)
            def _prefetch():
                pltpu.async_copy(
                    loc_hbm.at[pl.ds((g + 1) * SGROUP, SGROUP),
                               pl.ds(b0, LANES_PER_W)],
                    idx_v[1 - p], sem_i[1 - p])

            def ss_body(ss, carry):
                s = g * SGROUP + ss

                def q2_body(q2, carry2):
                    for qq in range(2):
                        q = 2 * q2 + qq

                        @pl.when((g > 0) | (ss > 0) | (q2 > 0))
                        def _drain():
                            pltpu.make_async_copy(
                                buf[qq],
                                out_hbm.at[0, :, pl.ds(0, 1), :, :],
                                sem_s[qq]).wait()

                        @plsc.parallel_loop(0, bi // 16)
                        def j_body(j):
                            idx16 = idx_v[p][ss, pl.ds(q * 128 + 16 * j, 16)]
                            base16 = idx16 * (c_dim + 1)

                            @plsc.parallel_loop(0, c_dim, unroll=16)
                            def c_body(c):
                                vals = plsc.load_gather(tab_v, [base16 + c])
                                buf[qq][c // 8, 0, c % 8,
                                        pl.ds(16 * j, 16)] = vals
                        pltpu.async_copy(
                            buf[qq],
                            out_hbm.at[s, :, pl.ds(NQ * w + q, 1), :, :],
                            sem_s[qq])
                    return carry2

                lax.fori_loop(0, NQ // 2, q2_body, 0)
                return carry

            lax.fori_loop(0, SGROUP, ss_body, 0)

        def pair_body(g2, carry):
            do_group(2 * g2, 0)
            do_group(2 * g2 + 1, 1)
            return carry

        lax.fori_loop(0, n_groups // 2, pair_body, 0)
        if n_groups % 2:
            do_group(jnp.int32(n_groups - 1), 0)

        for qb in range(2):
            pltpu.make_async_copy(buf[qb], out_hbm.at[0, :, pl.ds(0, 1), :, :],
                                  sem_s[qb]).wait()

    return k(tab_flat, loc_t)


def kernel(location, table):
    b_dim, s_dim = location.shape
    v, c_dim = table.shape
    loc_t = location.T.astype(jnp.int32)
    tab_flat = jnp.pad(table, ((0, -v % 8), (0, 1))).reshape(-1)
    out5 = _emb_lookup_t(loc_t, tab_flat, s_dim, c_dim, b_dim)
    return out5.transpose(2, 4, 0, 1, 3).reshape(b_dim, s_dim, c_dim)


# speedup vs baseline: 1.0034x; 1.0034x over previous
import functools

import jax
import jax.numpy as jnp
from jax import lax
from jax.experimental import pallas as pl
from jax.experimental.pallas import tpu as pltpu
from jax.experimental.pallas import tpu_sc as plsc

NUM_WORKERS = 32
LANES_PER_W = 512
SGROUP = 8
NQ = LANES_PER_W // 128


def _emb_lookup_t(loc_t, tab_flat, s_dim, c_dim, b_dim):
    ct, ci = c_dim // 8, 8
    bt, bi = b_dim // 128, 128
    n_groups = s_dim // SGROUP
    mesh = plsc.VectorSubcoreMesh(core_axis_name="c", subcore_axis_name="s")

    @functools.partial(
        pl.kernel,
        mesh=mesh,
        out_type=jax.ShapeDtypeStruct((s_dim, ct, bt, ci, bi), jnp.float32),
        scratch_types=[
            pltpu.VMEM(tab_flat.shape, jnp.float32),
            [pltpu.VMEM((SGROUP, LANES_PER_W), jnp.int32) for _ in range(2)],
            [pltpu.VMEM((ct, 1, ci, bi), jnp.float32) for _ in range(2)],
            pltpu.SemaphoreType.DMA,
            [pltpu.SemaphoreType.DMA for _ in range(2)],
            [pltpu.SemaphoreType.DMA for _ in range(2)],
        ],
        compiler_params=pltpu.CompilerParams(needs_layout_passes=False),
    )
    def k(tab_hbm, loc_hbm, out_hbm, tab_v, idx_v, buf, sem_t, sem_i, sem_s):
        w = lax.axis_index("s") * 2 + lax.axis_index("c")
        b0 = w * LANES_PER_W
        pltpu.async_copy(tab_hbm, tab_v, sem_t)
        pltpu.async_copy(loc_hbm.at[pl.ds(0, SGROUP), pl.ds(b0, LANES_PER_W)],
                         idx_v[0], sem_i[0])
        pltpu.make_async_copy(tab_hbm, tab_v, sem_t).wait()

        def do_group(g, p):
            pltpu.make_async_copy(
                loc_hbm.at[pl.ds(0, SGROUP), pl.ds(b0, LANES_PER_W)],
                idx_v[p], sem_i[p]).wait()

            @pl.when(g < n_groups - 1)
            def _prefetch():
                pltpu.async_copy(
                    loc_hbm.at[pl.ds((g + 1) * SGROUP, SGROUP),
                               pl.ds(b0, LANES_PER_W)],
                    idx_v[1 - p], sem_i[1 - p])

            def ss_body(ss, carry):
                s = g * SGROUP + ss

                def q2_body(q2, carry2):
                    for qq in range(2):
                        q = 2 * q2 + qq

                        @pl.when((g > 0) | (ss > 0) | (q2 > 0))
                        def _drain():
                            pltpu.make_async_copy(
                                buf[qq],
                                out_hbm.at[0, :, pl.ds(0, 1), :, :],
                                sem_s[qq]).wait()

                        def j_body(j, carry3):
                            idx16 = idx_v[p][ss, pl.ds(q * 128 + 16 * j, 16)]
                            base16 = idx16 * (c_dim + 1)

                            @plsc.parallel_loop(0, c_dim, unroll=16)
                            def c_body(c):
                                vals = plsc.load_gather(tab_v, [base16 + c])
                                buf[qq][c // 8, 0, c % 8,
                                        pl.ds(16 * j, 16)] = vals
                            return carry3

                        lax.fori_loop(0, bi // 16, j_body, 0)
                        pltpu.async_copy(
                            buf[qq],
                            out_hbm.at[s, :, pl.ds(NQ * w + q, 1), :, :],
                            sem_s[qq])
                    return carry2

                lax.fori_loop(0, NQ // 2, q2_body, 0)
                return carry

            lax.fori_loop(0, SGROUP, ss_body, 0)

        def pair_body(g2, carry):
            do_group(2 * g2, 0)
            do_group(2 * g2 + 1, 1)
            return carry

        lax.fori_loop(0, n_groups // 2, pair_body, 0)
        if n_groups % 2:
            do_group(jnp.int32(n_groups - 1), 0)

        for qb in range(2):
            pltpu.make_async_copy(buf[qb], out_hbm.at[0, :, pl.ds(0, 1), :, :],
                                  sem_s[qb]).wait()

    return k(tab_flat, loc_t)


def kernel(location, table):
    b_dim, s_dim = location.shape
    v, c_dim = table.shape
    loc_t = location.T.astype(jnp.int32)
    tab_flat = jnp.pad(table, ((0, -v % 8), (0, 1))).reshape(-1)
    out5 = _emb_lookup_t(loc_t, tab_flat, s_dim, c_dim, b_dim)
    return out5.transpose(2, 4, 0, 1, 3).reshape(b_dim, s_dim, c_dim)
